# Initial kernel scaffold; baseline (speedup 1.0000x reference)
import jax
import jax.numpy as jnp
from jax.experimental import pallas as pl


def kernel(x, codebook):
    raise NotImplementedError("write your pallas kernel here")


# speedup vs baseline: 1.2678x; 1.2678x over previous
import jax
import jax.numpy as jnp
from jax.experimental import pallas as pl

_VOCAB = 8192
_EMBED = 256
_M_TILE = 512


def _vq_body(x_ref, c_ref, o_ref):
    sim = jax.lax.dot_general(
        x_ref[...], c_ref[...],
        dimension_numbers=(((1,), (0,)), ((), ())),
        preferred_element_type=jnp.float32,
    )
    half = _VOCAB // 2
    s1, s2 = sim[:, :half], sim[:, half:]
    ids = jax.lax.broadcasted_iota(jnp.int32, s1.shape, 1)
    mx1 = jnp.max(s1, axis=1)
    i1 = jnp.min(jnp.where(s1 == mx1[:, None], ids, _VOCAB), axis=1)
    mx2 = jnp.max(s2, axis=1)
    i2 = jnp.min(jnp.where(s2 == mx2[:, None], ids, _VOCAB), axis=1) + half
    mx1r = mx1.astype(jnp.bfloat16).astype(jnp.float32)
    o_ref[...] = jnp.where(mx1r >= mx2, i1, i2)


def _l2_normalize(v, axis=-1, eps=1e-12):
    n = jnp.linalg.norm(v, axis=axis, keepdims=True)
    return v / jnp.maximum(n, eps)


def kernel(x, codebook):
    B, C, H, W = x.shape
    xf = jnp.transpose(x, (0, 2, 3, 1)).reshape(B * H * W, C)
    xn = _l2_normalize(xf, axis=1).astype(jnp.bfloat16)
    cnt = _l2_normalize(codebook, axis=1).astype(jnp.bfloat16).T
    n_tokens = B * H * W
    grid = (n_tokens // _M_TILE,)
    idx = pl.pallas_call(
        _vq_body,
        grid=grid,
        in_specs=[
            pl.BlockSpec((_M_TILE, _EMBED), lambda i: (i, 0)),
            pl.BlockSpec((_EMBED, _VOCAB), lambda i: (0, 0)),
        ],
        out_specs=pl.BlockSpec((_M_TILE,), lambda i: (i,)),
        out_shape=jax.ShapeDtypeStruct((n_tokens,), jnp.int32),
    )(xn, cnt)
    return idx


# speedup vs baseline: 1.4864x; 1.1724x over previous
import jax
import jax.numpy as jnp
from jax.experimental import pallas as pl

_VOCAB = 8192
_EMBED = 256
_EPS = 1e-12


def _vq_body(c_ref, x_ref, o_ref):
    sim = jax.lax.dot_general(
        c_ref[...], x_ref[0],
        dimension_numbers=(((1,), (0,)), ((), ())),
        preferred_element_type=jnp.float32,
    )
    half = _VOCAB // 2
    s1, s2 = sim[:half, :], sim[half:, :]
    ids = jax.lax.broadcasted_iota(jnp.int32, s1.shape, 0)
    mx1 = jnp.max(s1, axis=0)
    i1 = jnp.min(jnp.where(s1 == mx1[None, :], ids, _VOCAB), axis=0)
    mx2 = jnp.max(s2, axis=0)
    i2 = jnp.min(jnp.where(s2 == mx2[None, :], ids, _VOCAB), axis=0) + half
    mx1r = mx1.astype(jnp.bfloat16).astype(jnp.float32)
    o_ref[0, 0] = jnp.where(mx1r >= mx2, i1, i2)


def kernel(x, codebook):
    B, C, H, W = x.shape
    HW = H * W
    cn = codebook / jnp.maximum(
        jnp.linalg.norm(codebook, axis=1, keepdims=True), _EPS)
    cnb = cn.astype(jnp.bfloat16)
    x3 = x.reshape(B, C, HW)
    n = jnp.sqrt(jnp.sum(x3 * x3, axis=1)).reshape(B, 1, HW)
    xt = (x3 / jnp.maximum(n, _EPS)).astype(jnp.bfloat16)
    out = pl.pallas_call(
        _vq_body,
        grid=(B,),
        in_specs=[
            pl.BlockSpec((_VOCAB, _EMBED), lambda i: (0, 0)),
            pl.BlockSpec((1, _EMBED, HW), lambda i: (i, 0, 0)),
        ],
        out_specs=pl.BlockSpec((1, 1, HW), lambda i: (i, 0, 0)),
        out_shape=jax.ShapeDtypeStruct((B, 1, HW), jnp.int32),
    )(cnb, xt)
    return out.reshape(B * HW)


# speedup vs baseline: 1.5260x; 1.0266x over previous
import functools

import jax
import jax.numpy as jnp
from jax.experimental import pallas as pl
from jax.experimental.pallas import tpu as pltpu

_VOCAB = 8192
_EMBED = 256
_EPS = 1e-12
_ROWS = 8


def _scan_argmax(s_ref, base, rows, hw):
    nslab = rows // _ROWS

    def step(i, carry):
        acc_v, acc_s = carry
        s = s_ref[pl.ds(base + i * _ROWS, _ROWS), :]
        pred = s > acc_v
        acc_v = jnp.maximum(acc_v, s)
        acc_s = jnp.where(pred, jnp.full((_ROWS, hw), 0, jnp.int32) + i, acc_s)
        return acc_v, acc_s

    init = (jnp.full((_ROWS, hw), -jnp.inf, jnp.float32),
            jnp.zeros((_ROWS, hw), jnp.int32))
    acc_v, acc_s = jax.lax.fori_loop(0, nslab, step, init, unroll=8)
    sub = jax.lax.broadcasted_iota(jnp.int32, (_ROWS, hw), 0)
    gidx = acc_s * _ROWS + sub
    mx = jnp.max(acc_v, axis=0)
    idx = jnp.min(jnp.where(acc_v == mx[None, :], gidx, _VOCAB), axis=0)
    return mx, idx


def _vq_body(hw, c_ref, x_ref, o_ref, s_ref):
    s_ref[...] = jax.lax.dot_general(
        c_ref[...], x_ref[0],
        dimension_numbers=(((1,), (0,)), ((), ())),
        preferred_element_type=jnp.float32,
    )
    half = _VOCAB // 2
    mx1, i1 = _scan_argmax(s_ref, 0, half, hw)
    mx2, i2 = _scan_argmax(s_ref, half, half, hw)
    i2 = i2 + half
    mx1r = mx1.astype(jnp.bfloat16).astype(jnp.float32)
    o_ref[0, 0] = jnp.where(mx1r >= mx2, i1, i2)


def kernel(x, codebook):
    B, C, H, W = x.shape
    HW = H * W
    cn = codebook / jnp.maximum(
        jnp.linalg.norm(codebook, axis=1, keepdims=True), _EPS)
    cnb = cn.astype(jnp.bfloat16)
    x3 = x.reshape(B, C, HW)
    n = jnp.sqrt(jnp.sum(x3 * x3, axis=1)).reshape(B, 1, HW)
    xt = (x3 / jnp.maximum(n, _EPS)).astype(jnp.bfloat16)
    out = pl.pallas_call(
        functools.partial(_vq_body, HW),
        grid=(B,),
        in_specs=[
            pl.BlockSpec((_VOCAB, _EMBED), lambda i: (0, 0)),
            pl.BlockSpec((1, _EMBED, HW), lambda i: (i, 0, 0)),
        ],
        out_specs=pl.BlockSpec((1, 1, HW), lambda i: (i, 0, 0)),
        out_shape=jax.ShapeDtypeStruct((B, 1, HW), jnp.int32),
        scratch_shapes=[pltpu.VMEM((_VOCAB, HW), jnp.float32)],
    )(cnb, xt)
    return out.reshape(B * HW)
